# baseline (speedup 1.0000x reference)
import functools

import jax
import jax.numpy as jnp
from jax import lax
from jax.experimental import pallas as pl
from jax.experimental.pallas import tpu as pltpu
from jax.experimental.pallas import tpu_sc as plsc

_N_BUCKETS = 512
_SPB = 32
_TAU = 0.1
_ALPHA = 0.5

_G = 256
_P = 8


def _sc_gather(ids, tok_emb, tids2d, codebook):
  n = ids.shape[0]
  d = tok_emb.shape[1]
  info = plsc.get_sparse_core_info()
  nc, ns = info.num_cores, info.num_subcores
  nw = nc * ns
  per = n // nw
  ch = 128
  nch = per // ch

  mesh = plsc.VectorSubcoreMesh(core_axis_name="c", subcore_axis_name="s")

  @functools.partial(
      pl.kernel,
      out_type=(
          jax.ShapeDtypeStruct((n, d), jnp.float32),
          jax.ShapeDtypeStruct((n, d), jnp.float32),
          jax.ShapeDtypeStruct((n, 128), jnp.int32),
          jax.ShapeDtypeStruct((n,), jnp.int32),
      ),
      mesh=mesh,
      scratch_types=[
          pltpu.VMEM((nch, ch), jnp.int32),
          pltpu.VMEM((nch, ch), jnp.int32),
          pltpu.VMEM((per, d), jnp.float32),
          pltpu.VMEM((per, d), jnp.float32),
          pltpu.VMEM((per, 128), jnp.int32),
          pltpu.SemaphoreType.DMA,
      ],
  )
  def k(ids_hbm, emb_hbm, tids_hbm, cb_hbm,
        emb_o, anch_o, gt_o, bkt_o,
        ids_v, bkt_v, emb_v, anch_v, gt_v, sem):
    wid = lax.axis_index("s") * nc + lax.axis_index("c")
    base = wid * per
    for j in range(nch):
      pltpu.sync_copy(ids_hbm.at[pl.ds(base + j * ch, ch)], ids_v.at[j])
    for j in range(nch):
      for c in range(ch // 16):
        v = ids_v[j, pl.ds(c * 16, 16)]
        bkt_v[j, pl.ds(c * 16, 16)] = lax.rem(v, _N_BUCKETS)
    copies = []
    for j in range(nch):
      copies.append(pltpu.async_copy(
          emb_hbm.at[ids_v.at[j]], emb_v.at[pl.ds(j * ch, ch)], sem))
      copies.append(pltpu.async_copy(
          cb_hbm.at[bkt_v.at[j]], anch_v.at[pl.ds(j * ch, ch)], sem))
      copies.append(pltpu.async_copy(
          tids_hbm.at[bkt_v.at[j]], gt_v.at[pl.ds(j * ch, ch)], sem))
    for cp in copies:
      cp.wait()
    pltpu.sync_copy(emb_v, emb_o.at[pl.ds(base, per)])
    pltpu.sync_copy(anch_v, anch_o.at[pl.ds(base, per)])
    pltpu.sync_copy(gt_v, gt_o.at[pl.ds(base, per)])
    for j in range(nch):
      pltpu.sync_copy(bkt_v.at[j], bkt_o.at[pl.ds(base + j * ch, ch)])

  return k(ids, tok_emb, tids2d, codebook)


_NBUF = 3


def _tc_body(bkt_ref, emb_ref, pe_ref, ids_ref, gt_ref, anc_ref,
             keys_ref, vals_ref, out_ref, kbuf, vbuf, u_ref, sems):
  i = pl.program_id(0)
  h = emb_ref[...] + pe_ref[...]
  qn = h * lax.rsqrt(jnp.maximum(jnp.sum(h * h, -1, keepdims=True), 1e-24))
  u = _ALPHA * qn + (1.0 - _ALPHA) * anc_ref[...]
  u = u * lax.rsqrt(jnp.maximum(jnp.sum(u * u, -1, keepdims=True), 1e-24))
  u_ref[...] = u

  w = _P * _SPB
  ng = _G // _P
  col = lax.broadcasted_iota(jnp.int32, (_P, w), 1)
  row = lax.broadcasted_iota(jnp.int32, (_P, w), 0)
  bd = (col // _SPB) == row

  def issue(g, slot):
    t0 = i * _G + g * _P
    descs = []
    for j in range(_P):
      b = bkt_ref[t0 + j]
      b0 = b * _SPB
      descs.append(pltpu.make_async_copy(
          keys_ref.at[pl.ds(b0, _SPB), :],
          kbuf.at[slot, pl.ds(j * _SPB, _SPB), :], sems.at[slot]))
      descs.append(pltpu.make_async_copy(
          vals_ref.at[pl.ds(b0, _SPB), :],
          vbuf.at[slot, pl.ds(j * _SPB, _SPB), :], sems.at[slot]))
    for d in descs:
      d.start()
    return descs

  pending = {}
  pending[0] = issue(0, 0)
  for g in range(ng):
    slot = g % _NBUF
    if g + 1 < ng:
      pending[g + 1] = issue(g + 1, (g + 1) % _NBUF)
    for d in pending.pop(g):
      d.wait()
    q = u_ref[pl.ds(g * _P, _P), :]
    scores = lax.dot_general(
        q, kbuf[slot], (((1,), (1,)), ((), ())),
        precision=lax.Precision.HIGHEST,
        preferred_element_type=jnp.float32)
    tc8 = gt_ref[pl.ds(g * _P, _P), :]
    ttile = jnp.concatenate([tc8, tc8], axis=1)
    idsp = ids_ref[pl.ds(g * _P, _P), :]
    match = jnp.where(bd & (ttile == idsp), 1.0, 0.0).astype(jnp.float32)
    msum = jnp.sum(match, -1, keepdims=True)
    sc = jnp.where(bd, scores * (1.0 / _TAU), -1e30)
    m = jnp.max(sc, -1, keepdims=True)
    e = jnp.exp(sc - m)
    psoft = e / jnp.sum(e, -1, keepdims=True)
    probs = jnp.where(msum > 0, match / (msum + 1e-9), psoft)
    val = lax.dot_general(
        probs, vbuf[slot], (((1,), (0,)), ((), ())),
        precision=lax.Precision.HIGHEST,
        preferred_element_type=jnp.float32)
    out_ref[pl.ds(g * _P, _P), :] = val


def _tc_combine(buckets, emb, pe, ids2, gtids_g, anchors, keys, vals, t):
  n, d = emb.shape
  w = _P * _SPB
  grid = (n // _G,)
  spec = pltpu.PrefetchScalarGridSpec(
      num_scalar_prefetch=1,
      grid=grid,
      in_specs=[
          pl.BlockSpec((_G, d), lambda i, b: (i, 0)),
          pl.BlockSpec((_G, d), lambda i, b: (i % (t // _G), 0)),
          pl.BlockSpec((_G, 1), lambda i, b: (i, 0)),
          pl.BlockSpec((_G, 128), lambda i, b: (i, 0)),
          pl.BlockSpec((_G, d), lambda i, b: (i, 0)),
          pl.BlockSpec(keys.shape, lambda i, b: (0, 0)),
          pl.BlockSpec(vals.shape, lambda i, b: (0, 0)),
      ],
      out_specs=pl.BlockSpec((_G, d), lambda i, b: (i, 0)),
      scratch_shapes=[
          pltpu.VMEM((_NBUF, w, d), jnp.float32),
          pltpu.VMEM((_NBUF, w, d), jnp.float32),
          pltpu.VMEM((_G, d), jnp.float32),
          pltpu.SemaphoreType.DMA((_NBUF,)),
      ],
  )
  return pl.pallas_call(
      _tc_body,
      grid_spec=spec,
      out_shape=jax.ShapeDtypeStruct((n, d), jnp.float32),
      compiler_params=pltpu.CompilerParams(
          dimension_semantics=("arbitrary",)),
  )(buckets, emb, pe, ids2, gtids_g, anchors, keys, vals)


def kernel(input_ids, tok_emb, slot_keys, slot_values, centroid_codebook,
           pe, slot_tids):
  b, t = input_ids.shape
  d = tok_emb.shape[1]
  n = b * t
  ids = input_ids.reshape(n).astype(jnp.int32)
  tids_tiled = jnp.tile(
      slot_tids.astype(jnp.int32).reshape(_N_BUCKETS, _SPB), (1, 4))

  emb, anchors, gtids, buckets = _sc_gather(ids, tok_emb, tids_tiled,
                                            centroid_codebook)
  ids2 = ids.reshape(n, 1)
  out = _tc_combine(buckets, emb, pe, ids2, gtids, anchors,
                    slot_keys, slot_values, t)
  return out.reshape(b, t, d)


# speedup vs baseline: 1.6635x; 1.6635x over previous
import functools

import jax
import jax.numpy as jnp
from jax import lax
from jax.experimental import pallas as pl
from jax.experimental.pallas import tpu as pltpu
from jax.experimental.pallas import tpu_sc as plsc

_N_BUCKETS = 512
_SPB = 32
_TAU = 0.1
_ALPHA = 0.5

_G = 128
_P = 8


def _sc_gather(ids, tok_emb, tids2d, codebook):
  n = ids.shape[0]
  d = tok_emb.shape[1]
  info = plsc.get_sparse_core_info()
  nc, ns = info.num_cores, info.num_subcores
  nw = nc * ns
  per = n // nw
  ch = 128
  nch = per // ch

  mesh = plsc.VectorSubcoreMesh(core_axis_name="c", subcore_axis_name="s")

  @functools.partial(
      pl.kernel,
      out_type=(
          jax.ShapeDtypeStruct((n, d), jnp.float32),
          jax.ShapeDtypeStruct((n, d), jnp.float32),
          jax.ShapeDtypeStruct((n, 128), jnp.int32),
          jax.ShapeDtypeStruct((n,), jnp.int32),
      ),
      mesh=mesh,
      scratch_types=[
          pltpu.VMEM((nch, ch), jnp.int32),
          pltpu.VMEM((nch, ch), jnp.int32),
          pltpu.VMEM((per, d), jnp.float32),
          pltpu.VMEM((per, d), jnp.float32),
          pltpu.VMEM((per, 128), jnp.int32),
          pltpu.SemaphoreType.DMA,
      ],
  )
  def k(ids_hbm, emb_hbm, tids_hbm, cb_hbm,
        emb_o, anch_o, gt_o, bkt_o,
        ids_v, bkt_v, emb_v, anch_v, gt_v, sem):
    wid = lax.axis_index("s") * nc + lax.axis_index("c")
    base = wid * per
    for j in range(nch):
      pltpu.sync_copy(ids_hbm.at[pl.ds(base + j * ch, ch)], ids_v.at[j])
    for j in range(nch):
      for c in range(ch // 16):
        v = ids_v[j, pl.ds(c * 16, 16)]
        bkt_v[j, pl.ds(c * 16, 16)] = lax.rem(v, _N_BUCKETS)
    copies = []
    for j in range(nch):
      copies.append(pltpu.async_copy(
          emb_hbm.at[ids_v.at[j]], emb_v.at[pl.ds(j * ch, ch)], sem))
      copies.append(pltpu.async_copy(
          cb_hbm.at[bkt_v.at[j]], anch_v.at[pl.ds(j * ch, ch)], sem))
      copies.append(pltpu.async_copy(
          tids_hbm.at[bkt_v.at[j]], gt_v.at[pl.ds(j * ch, ch)], sem))
    for cp in copies:
      cp.wait()
    pltpu.sync_copy(emb_v, emb_o.at[pl.ds(base, per)])
    pltpu.sync_copy(anch_v, anch_o.at[pl.ds(base, per)])
    pltpu.sync_copy(gt_v, gt_o.at[pl.ds(base, per)])
    for j in range(nch):
      pltpu.sync_copy(bkt_v.at[j], bkt_o.at[pl.ds(base + j * ch, ch)])

  return k(ids, tok_emb, tids2d, codebook)


def _tc_body(bkt_ref, emb_ref, pe_ref, ids_ref, gt_ref, anc_ref,
             keys_ref, vals_ref, out_ref, kall, vall, u_ref):
  i = pl.program_id(0)
  h = emb_ref[...] + pe_ref[...]
  qn = h * lax.rsqrt(jnp.maximum(jnp.sum(h * h, -1, keepdims=True), 1e-24))
  u = _ALPHA * qn + (1.0 - _ALPHA) * anc_ref[...]
  u = u * lax.rsqrt(jnp.maximum(jnp.sum(u * u, -1, keepdims=True), 1e-24))
  u_ref[...] = u

  w = _P * _SPB
  ng = _G // _P
  col = lax.broadcasted_iota(jnp.int32, (_P, w), 1)
  row = lax.broadcasted_iota(jnp.int32, (_P, w), 0)
  bd = (col // _SPB) == row

  bs = [bkt_ref[i * _G + t] * _SPB for t in range(_G)]
  for g in range(ng):
    for j in range(_P):
      kall[g, pl.ds(j * _SPB, _SPB), :] = \
          keys_ref[pl.ds(bs[g * _P + j], _SPB), :]
  for g in range(ng):
    for j in range(_P):
      vall[g, pl.ds(j * _SPB, _SPB), :] = \
          vals_ref[pl.ds(bs[g * _P + j], _SPB), :]

  for g in range(ng):
    q = u_ref[pl.ds(g * _P, _P), :]
    scores = lax.dot_general(
        q, kall[g], (((1,), (1,)), ((), ())),
        precision=lax.Precision.HIGHEST,
        preferred_element_type=jnp.float32)
    tc8 = gt_ref[pl.ds(g * _P, _P), :]
    ttile = jnp.concatenate([tc8, tc8], axis=1)
    idsp = ids_ref[pl.ds(g * _P, _P), :]
    match = jnp.where(bd & (ttile == idsp), 1.0, 0.0).astype(jnp.float32)
    msum = jnp.sum(match, -1, keepdims=True)
    sc = jnp.where(bd, scores * (1.0 / _TAU), -1e30)
    m = jnp.max(sc, -1, keepdims=True)
    e = jnp.exp(sc - m)
    psoft = e / jnp.sum(e, -1, keepdims=True)
    probs = jnp.where(msum > 0, match / (msum + 1e-9), psoft)
    val = lax.dot_general(
        probs, vall[g], (((1,), (0,)), ((), ())),
        precision=lax.Precision.HIGHEST,
        preferred_element_type=jnp.float32)
    out_ref[pl.ds(g * _P, _P), :] = val


def _tc_combine(buckets, emb, pe, ids2, gtids_g, anchors, keys, vals, t):
  n, d = emb.shape
  w = _P * _SPB
  grid = (n // _G,)
  spec = pltpu.PrefetchScalarGridSpec(
      num_scalar_prefetch=1,
      grid=grid,
      in_specs=[
          pl.BlockSpec((_G, d), lambda i, b: (i, 0)),
          pl.BlockSpec((_G, d), lambda i, b: (i % (t // _G), 0)),
          pl.BlockSpec((_G, 1), lambda i, b: (i, 0)),
          pl.BlockSpec((_G, 128), lambda i, b: (i, 0)),
          pl.BlockSpec((_G, d), lambda i, b: (i, 0)),
          pl.BlockSpec(keys.shape, lambda i, b: (0, 0)),
          pl.BlockSpec(vals.shape, lambda i, b: (0, 0)),
      ],
      out_specs=pl.BlockSpec((_G, d), lambda i, b: (i, 0)),
      scratch_shapes=[
          pltpu.VMEM((_G // _P, w, d), jnp.float32),
          pltpu.VMEM((_G // _P, w, d), jnp.float32),
          pltpu.VMEM((_G, d), jnp.float32),
      ],
  )
  return pl.pallas_call(
      _tc_body,
      grid_spec=spec,
      out_shape=jax.ShapeDtypeStruct((n, d), jnp.float32),
      compiler_params=pltpu.CompilerParams(
          dimension_semantics=("arbitrary",)),
  )(buckets, emb, pe, ids2, gtids_g, anchors, keys, vals)


def kernel(input_ids, tok_emb, slot_keys, slot_values, centroid_codebook,
           pe, slot_tids):
  b, t = input_ids.shape
  d = tok_emb.shape[1]
  n = b * t
  ids = input_ids.reshape(n).astype(jnp.int32)
  tids_tiled = jnp.tile(
      slot_tids.astype(jnp.int32).reshape(_N_BUCKETS, _SPB), (1, 4))

  emb, anchors, gtids, buckets = _sc_gather(ids, tok_emb, tids_tiled,
                                            centroid_codebook)
  ids2 = ids.reshape(n, 1)
  out = _tc_combine(buckets, emb, pe, ids2, gtids, anchors,
                    slot_keys, slot_values, t)
  return out.reshape(b, t, d)


# speedup vs baseline: 3.5234x; 2.1181x over previous
import functools

import jax
import jax.numpy as jnp
from jax import lax
from jax.experimental import pallas as pl
from jax.experimental.pallas import tpu as pltpu
from jax.experimental.pallas import tpu_sc as plsc

_N_BUCKETS = 512
_SPB = 32
_TAU = 0.1
_ALPHA = 0.5

_G = 128
_P = 64


def _sc_gather(ids, tok_emb, tids2d, codebook):
  n = ids.shape[0]
  d = tok_emb.shape[1]
  info = plsc.get_sparse_core_info()
  nc, ns = info.num_cores, info.num_subcores
  nw = nc * ns
  per = n // nw
  ch = 128
  nch = per // ch

  mesh = plsc.VectorSubcoreMesh(core_axis_name="c", subcore_axis_name="s")

  @functools.partial(
      pl.kernel,
      out_type=(
          jax.ShapeDtypeStruct((n, d), jnp.float32),
          jax.ShapeDtypeStruct((n, d), jnp.float32),
          jax.ShapeDtypeStruct((n, 128), jnp.int32),
          jax.ShapeDtypeStruct((n,), jnp.int32),
      ),
      mesh=mesh,
      scratch_types=[
          pltpu.VMEM((nch, ch), jnp.int32),
          pltpu.VMEM((nch, ch), jnp.int32),
          pltpu.VMEM((per, d), jnp.float32),
          pltpu.VMEM((per, d), jnp.float32),
          pltpu.VMEM((per, 128), jnp.int32),
          pltpu.SemaphoreType.DMA,
      ],
  )
  def k(ids_hbm, emb_hbm, tids_hbm, cb_hbm,
        emb_o, anch_o, gt_o, bkt_o,
        ids_v, bkt_v, emb_v, anch_v, gt_v, sem):
    wid = lax.axis_index("s") * nc + lax.axis_index("c")
    base = wid * per
    for j in range(nch):
      pltpu.sync_copy(ids_hbm.at[pl.ds(base + j * ch, ch)], ids_v.at[j])
    for j in range(nch):
      for c in range(ch // 16):
        v = ids_v[j, pl.ds(c * 16, 16)]
        bkt_v[j, pl.ds(c * 16, 16)] = lax.rem(v, _N_BUCKETS)
    copies = []
    for j in range(nch):
      copies.append(pltpu.async_copy(
          emb_hbm.at[ids_v.at[j]], emb_v.at[pl.ds(j * ch, ch)], sem))
      copies.append(pltpu.async_copy(
          cb_hbm.at[bkt_v.at[j]], anch_v.at[pl.ds(j * ch, ch)], sem))
      copies.append(pltpu.async_copy(
          tids_hbm.at[bkt_v.at[j]], gt_v.at[pl.ds(j * ch, ch)], sem))
    for cp in copies:
      cp.wait()
    pltpu.sync_copy(emb_v, emb_o.at[pl.ds(base, per)])
    pltpu.sync_copy(anch_v, anch_o.at[pl.ds(base, per)])
    pltpu.sync_copy(gt_v, gt_o.at[pl.ds(base, per)])
    for j in range(nch):
      pltpu.sync_copy(bkt_v.at[j], bkt_o.at[pl.ds(base + j * ch, ch)])

  return k(ids, tok_emb, tids2d, codebook)


def _tc_body(bkt_ref, emb_ref, pe_ref, ids_ref, gt_ref, anc_ref,
             keys_ref, vals_ref, out_ref, kall, vall, u_ref):
  i = pl.program_id(0)
  h = emb_ref[...] + pe_ref[...]
  qn = h * lax.rsqrt(jnp.maximum(jnp.sum(h * h, -1, keepdims=True), 1e-24))
  u = _ALPHA * qn + (1.0 - _ALPHA) * anc_ref[...]
  u = u * lax.rsqrt(jnp.maximum(jnp.sum(u * u, -1, keepdims=True), 1e-24))
  u_ref[...] = u

  w = _P * _SPB
  ng = _G // _P
  col = lax.broadcasted_iota(jnp.int32, (_P, w), 1)
  row = lax.broadcasted_iota(jnp.int32, (_P, w), 0)
  bd = (col // _SPB) == row

  bs = [bkt_ref[i * _G + t] * _SPB for t in range(_G)]
  for g in range(ng):
    for j in range(_P):
      kall[g, pl.ds(j * _SPB, _SPB), :] = \
          keys_ref[pl.ds(bs[g * _P + j], _SPB), :]
  for g in range(ng):
    for j in range(_P):
      vall[g, pl.ds(j * _SPB, _SPB), :] = \
          vals_ref[pl.ds(bs[g * _P + j], _SPB), :]

  for g in range(ng):
    q = u_ref[pl.ds(g * _P, _P), :]
    scores = lax.dot_general(
        q, kall[g], (((1,), (1,)), ((), ())),
        precision=lax.Precision.HIGHEST,
        preferred_element_type=jnp.float32)
    tc8 = gt_ref[pl.ds(g * _P, _P), :]
    ttile = jnp.concatenate([tc8] * (w // 128), axis=1)
    idsp = ids_ref[pl.ds(g * _P, _P), :]
    match = jnp.where(bd & (ttile == idsp), 1.0, 0.0).astype(jnp.float32)
    msum = jnp.sum(match, -1, keepdims=True)
    sc = jnp.where(bd, scores * (1.0 / _TAU), -1e30)
    m = jnp.max(sc, -1, keepdims=True)
    e = jnp.exp(sc - m)
    psoft = e / jnp.sum(e, -1, keepdims=True)
    probs = jnp.where(msum > 0, match / (msum + 1e-9), psoft)
    val = lax.dot_general(
        probs, vall[g], (((1,), (0,)), ((), ())),
        precision=lax.Precision.HIGHEST,
        preferred_element_type=jnp.float32)
    out_ref[pl.ds(g * _P, _P), :] = val


def _tc_combine(buckets, emb, pe, ids2, gtids_g, anchors, keys, vals, t):
  n, d = emb.shape
  w = _P * _SPB
  grid = (n // _G,)
  spec = pltpu.PrefetchScalarGridSpec(
      num_scalar_prefetch=1,
      grid=grid,
      in_specs=[
          pl.BlockSpec((_G, d), lambda i, b: (i, 0)),
          pl.BlockSpec((_G, d), lambda i, b: (i % (t // _G), 0)),
          pl.BlockSpec((_G, 1), lambda i, b: (i, 0)),
          pl.BlockSpec((_G, 128), lambda i, b: (i, 0)),
          pl.BlockSpec((_G, d), lambda i, b: (i, 0)),
          pl.BlockSpec(keys.shape, lambda i, b: (0, 0)),
          pl.BlockSpec(vals.shape, lambda i, b: (0, 0)),
      ],
      out_specs=pl.BlockSpec((_G, d), lambda i, b: (i, 0)),
      scratch_shapes=[
          pltpu.VMEM((_G // _P, w, d), jnp.float32),
          pltpu.VMEM((_G // _P, w, d), jnp.float32),
          pltpu.VMEM((_G, d), jnp.float32),
      ],
  )
  return pl.pallas_call(
      _tc_body,
      grid_spec=spec,
      out_shape=jax.ShapeDtypeStruct((n, d), jnp.float32),
      compiler_params=pltpu.CompilerParams(
          dimension_semantics=("arbitrary",)),
  )(buckets, emb, pe, ids2, gtids_g, anchors, keys, vals)


def kernel(input_ids, tok_emb, slot_keys, slot_values, centroid_codebook,
           pe, slot_tids):
  b, t = input_ids.shape
  d = tok_emb.shape[1]
  n = b * t
  ids = input_ids.reshape(n).astype(jnp.int32)
  tids_tiled = jnp.tile(
      slot_tids.astype(jnp.int32).reshape(_N_BUCKETS, _SPB), (1, 4))

  emb, anchors, gtids, buckets = _sc_gather(ids, tok_emb, tids_tiled,
                                            centroid_codebook)
  ids2 = ids.reshape(n, 1)
  out = _tc_combine(buckets, emb, pe, ids2, gtids, anchors,
                    slot_keys, slot_values, t)
  return out.reshape(b, t, d)


# speedup vs baseline: 7.8459x; 2.2268x over previous
import functools

import jax
import jax.numpy as jnp
from jax import lax
from jax.experimental import pallas as pl
from jax.experimental.pallas import tpu as pltpu
from jax.experimental.pallas import tpu_sc as plsc

_N_BUCKETS = 512
_SPB = 32
_TAU = 0.1
_ALPHA = 0.5

_G = 128
_P = 64


def _sc_gather(ids, tok_emb, tids2d, codebook):
  n = ids.shape[0]
  d = tok_emb.shape[1]
  info = plsc.get_sparse_core_info()
  nc, ns = info.num_cores, info.num_subcores
  nw = nc * ns
  per = n // nw
  ch = 128
  nch = per // ch

  mesh = plsc.VectorSubcoreMesh(core_axis_name="c", subcore_axis_name="s")

  @functools.partial(
      pl.kernel,
      out_type=(
          jax.ShapeDtypeStruct((n, d), jnp.float32),
          jax.ShapeDtypeStruct((n, d), jnp.float32),
          jax.ShapeDtypeStruct((n, 128), jnp.int32),
          jax.ShapeDtypeStruct((n,), jnp.int32),
      ),
      mesh=mesh,
      scratch_types=[
          pltpu.VMEM((nch, ch), jnp.int32),
          pltpu.VMEM((nch, ch), jnp.int32),
          pltpu.VMEM((per, d), jnp.float32),
          pltpu.VMEM((per, d), jnp.float32),
          pltpu.VMEM((per, 128), jnp.int32),
          pltpu.SemaphoreType.DMA,
      ],
  )
  def k(ids_hbm, emb_hbm, tids_hbm, cb_hbm,
        emb_o, anch_o, gt_o, bkt_o,
        ids_v, bkt_v, emb_v, anch_v, gt_v, sem):
    wid = lax.axis_index("s") * nc + lax.axis_index("c")
    base = wid * per
    for j in range(nch):
      pltpu.sync_copy(ids_hbm.at[pl.ds(base + j * ch, ch)], ids_v.at[j])
    for j in range(nch):
      for c in range(ch // 16):
        v = ids_v[j, pl.ds(c * 16, 16)]
        bkt_v[j, pl.ds(c * 16, 16)] = lax.rem(v, _N_BUCKETS)
    copies = []
    for j in range(nch):
      copies.append(pltpu.async_copy(
          emb_hbm.at[ids_v.at[j]], emb_v.at[pl.ds(j * ch, ch)], sem))
      copies.append(pltpu.async_copy(
          cb_hbm.at[bkt_v.at[j]], anch_v.at[pl.ds(j * ch, ch)], sem))
      copies.append(pltpu.async_copy(
          tids_hbm.at[bkt_v.at[j]], gt_v.at[pl.ds(j * ch, ch)], sem))
    for cp in copies:
      cp.wait()
    pltpu.sync_copy(emb_v, emb_o.at[pl.ds(base, per)])
    pltpu.sync_copy(anch_v, anch_o.at[pl.ds(base, per)])
    pltpu.sync_copy(gt_v, gt_o.at[pl.ds(base, per)])
    for j in range(nch):
      pltpu.sync_copy(bkt_v.at[j], bkt_o.at[pl.ds(base + j * ch, ch)])

  return k(ids, tok_emb, tids2d, codebook)


def _tc_body(bkt_ref, emb_ref, pe_ref, ids_ref, gt_ref, anc_ref,
             keys_ref, vals_ref, out_ref, kall, vall, u_ref):
  i = pl.program_id(0)
  h = emb_ref[...] + pe_ref[...]
  qn = h * lax.rsqrt(jnp.maximum(jnp.sum(h * h, -1, keepdims=True), 1e-24))
  u = _ALPHA * qn + (1.0 - _ALPHA) * anc_ref[...]
  u = u * lax.rsqrt(jnp.maximum(jnp.sum(u * u, -1, keepdims=True), 1e-24))
  u_ref[...] = u

  w = _P * _SPB
  ng = _G // _P
  col = lax.broadcasted_iota(jnp.int32, (_P, w), 1)
  row = lax.broadcasted_iota(jnp.int32, (_P, w), 0)
  bd = (col // _SPB) == row

  bs = [bkt_ref[i * _G + t] * _SPB for t in range(_G)]
  for g in range(ng):
    for j in range(_P):
      kall[g, pl.ds(j * _SPB, _SPB), :] = \
          keys_ref[pl.ds(bs[g * _P + j], _SPB), :]
  for g in range(ng):
    for j in range(_P):
      vall[g, pl.ds(j * _SPB, _SPB), :] = \
          vals_ref[pl.ds(bs[g * _P + j], _SPB), :]

  for g in range(ng):
    q = u_ref[pl.ds(g * _P, _P), :]
    scores = lax.dot_general(
        q, kall[g], (((1,), (1,)), ((), ())),
        precision=lax.Precision.DEFAULT,
        preferred_element_type=jnp.float32)
    tc8 = gt_ref[pl.ds(g * _P, _P), :]
    ttile = jnp.concatenate([tc8] * (w // 128), axis=1)
    idsp = ids_ref[pl.ds(g * _P, _P), :]
    match = jnp.where(bd & (ttile == idsp), 1.0, 0.0).astype(jnp.float32)
    msum = jnp.sum(match, -1, keepdims=True)
    sc = jnp.where(bd, scores * (1.0 / _TAU), -1e30)
    m = jnp.max(sc, -1, keepdims=True)
    e = jnp.exp(sc - m)
    psoft = e / jnp.sum(e, -1, keepdims=True)
    probs = jnp.where(msum > 0, match / (msum + 1e-9), psoft)
    val = lax.dot_general(
        probs, vall[g], (((1,), (0,)), ((), ())),
        precision=lax.Precision.DEFAULT,
        preferred_element_type=jnp.float32)
    out_ref[pl.ds(g * _P, _P), :] = val


def _tc_combine(buckets, emb, pe, ids2, gtids_g, anchors, keys, vals, t):
  n, d = emb.shape
  w = _P * _SPB
  grid = (n // _G,)
  spec = pltpu.PrefetchScalarGridSpec(
      num_scalar_prefetch=1,
      grid=grid,
      in_specs=[
          pl.BlockSpec((_G, d), lambda i, b: (i, 0)),
          pl.BlockSpec((_G, d), lambda i, b: (i % (t // _G), 0)),
          pl.BlockSpec((_G, 1), lambda i, b: (i, 0)),
          pl.BlockSpec((_G, 128), lambda i, b: (i, 0)),
          pl.BlockSpec((_G, d), lambda i, b: (i, 0)),
          pl.BlockSpec(keys.shape, lambda i, b: (0, 0)),
          pl.BlockSpec(vals.shape, lambda i, b: (0, 0)),
      ],
      out_specs=pl.BlockSpec((_G, d), lambda i, b: (i, 0)),
      scratch_shapes=[
          pltpu.VMEM((_G // _P, w, d), jnp.float32),
          pltpu.VMEM((_G // _P, w, d), jnp.float32),
          pltpu.VMEM((_G, d), jnp.float32),
      ],
  )
  return pl.pallas_call(
      _tc_body,
      grid_spec=spec,
      out_shape=jax.ShapeDtypeStruct((n, d), jnp.float32),
      compiler_params=pltpu.CompilerParams(
          dimension_semantics=("arbitrary",)),
  )(buckets, emb, pe, ids2, gtids_g, anchors, keys, vals)


def kernel(input_ids, tok_emb, slot_keys, slot_values, centroid_codebook,
           pe, slot_tids):
  b, t = input_ids.shape
  d = tok_emb.shape[1]
  n = b * t
  ids = input_ids.reshape(n).astype(jnp.int32)
  tids_tiled = jnp.tile(
      slot_tids.astype(jnp.int32).reshape(_N_BUCKETS, _SPB), (1, 4))

  emb, anchors, gtids, buckets = _sc_gather(ids, tok_emb, tids_tiled,
                                            centroid_codebook)
  ids2 = ids.reshape(n, 1)
  out = _tc_combine(buckets, emb, pe, ids2, gtids, anchors,
                    slot_keys, slot_values, t)
  return out.reshape(b, t, d)


# speedup vs baseline: 8.9637x; 1.1425x over previous
import functools

import jax
import jax.numpy as jnp
from jax import lax
from jax.experimental import pallas as pl
from jax.experimental.pallas import tpu as pltpu
from jax.experimental.pallas import tpu_sc as plsc

_N_BUCKETS = 512
_SPB = 32
_TAU = 0.1
_ALPHA = 0.5

_G = 512
_P = 128


def _sc_gather(ids, tok_emb):
  n = ids.shape[0]
  d = tok_emb.shape[1]
  info = plsc.get_sparse_core_info()
  nc, ns = info.num_cores, info.num_subcores
  nw = nc * ns
  per = n // nw
  ch = 128
  nch = per // ch

  mesh = plsc.VectorSubcoreMesh(core_axis_name="c", subcore_axis_name="s")

  @functools.partial(
      pl.kernel,
      out_type=jax.ShapeDtypeStruct((n, d), jnp.float32),
      mesh=mesh,
      scratch_types=[
          pltpu.VMEM((nch, ch), jnp.int32),
          pltpu.VMEM((per, d), jnp.float32),
          pltpu.SemaphoreType.DMA,
      ],
  )
  def k(ids_hbm, emb_hbm, emb_o, ids_v, emb_v, sem):
    wid = lax.axis_index("s") * nc + lax.axis_index("c")
    base = wid * per
    for j in range(nch):
      pltpu.sync_copy(ids_hbm.at[pl.ds(base + j * ch, ch)], ids_v.at[j])
    copies = []
    for j in range(nch):
      copies.append(pltpu.async_copy(
          emb_hbm.at[ids_v.at[j]], emb_v.at[pl.ds(j * ch, ch)], sem))
    for cp in copies:
      cp.wait()
    pltpu.sync_copy(emb_v, emb_o.at[pl.ds(base, per)])

  return k(ids, tok_emb)


def _tc_body(ids_smem, emb_ref, pe_ref, ids_ref, cb_ref, tids_ref,
             keys_ref, vals_ref, out_ref, kall, vall, u_ref, gt_ref):
  i = pl.program_id(0)
  h = emb_ref[...] + pe_ref[...]
  qn = h * lax.rsqrt(jnp.maximum(jnp.sum(h * h, -1, keepdims=True), 1e-24))

  idsv = ids_ref[...]
  bktv = idsv & (_N_BUCKETS - 1)
  onehot = jnp.where(
      bktv == lax.broadcasted_iota(jnp.int32, (_G, _N_BUCKETS), 1),
      1.0, 0.0).astype(jnp.float32)
  anchors = lax.dot_general(
      onehot, cb_ref[...], (((1,), (0,)), ((), ())),
      preferred_element_type=jnp.float32)
  u = _ALPHA * qn + (1.0 - _ALPHA) * anchors
  u = u * lax.rsqrt(jnp.maximum(jnp.sum(u * u, -1, keepdims=True), 1e-24))
  u_ref[...] = u
  gt_ref[...] = lax.dot_general(
      onehot, tids_ref[...], (((1,), (0,)), ((), ())),
      precision=lax.Precision.HIGHEST,
      preferred_element_type=jnp.float32)
  ids_f = idsv.astype(jnp.float32)

  w = _P * _SPB
  ng = _G // _P
  col = lax.broadcasted_iota(jnp.int32, (_P, w), 1)
  row = lax.broadcasted_iota(jnp.int32, (_P, w), 0)
  bd = (col // _SPB) == row

  bs = [(ids_smem[i * _G + t] & (_N_BUCKETS - 1)) * _SPB
        for t in range(_G)]
  for g in range(ng):
    for j in range(_P):
      kall[g, pl.ds(j * _SPB, _SPB), :] = \
          keys_ref[pl.ds(bs[g * _P + j], _SPB), :]
  for g in range(ng):
    for j in range(_P):
      vall[g, pl.ds(j * _SPB, _SPB), :] = \
          vals_ref[pl.ds(bs[g * _P + j], _SPB), :]

  for g in range(ng):
    q = u_ref[pl.ds(g * _P, _P), :]
    scores = lax.dot_general(
        q, kall[g], (((1,), (1,)), ((), ())),
        preferred_element_type=jnp.float32)
    tcp = gt_ref[pl.ds(g * _P, _P), :]
    ttile = jnp.concatenate([tcp] * (w // 128), axis=1)
    idsp = lax.slice(ids_f, (g * _P, 0), ((g + 1) * _P, 1))
    match = jnp.where(bd & (ttile == idsp), 1.0, 0.0).astype(jnp.float32)
    msum = jnp.sum(match, -1, keepdims=True)
    sc = jnp.where(bd, scores * (1.0 / _TAU), -1e30)
    m = jnp.max(sc, -1, keepdims=True)
    e = jnp.exp(sc - m)
    psoft = e / jnp.sum(e, -1, keepdims=True)
    probs = jnp.where(msum > 0, match / (msum + 1e-9), psoft)
    val = lax.dot_general(
        probs, vall[g], (((1,), (0,)), ((), ())),
        preferred_element_type=jnp.float32)
    out_ref[pl.ds(g * _P, _P), :] = val


def _tc_combine(ids, emb, pe, ids2, cb, tids_f, keys, vals, t):
  n, d = emb.shape
  w = _P * _SPB
  grid = (n // _G,)
  spec = pltpu.PrefetchScalarGridSpec(
      num_scalar_prefetch=1,
      grid=grid,
      in_specs=[
          pl.BlockSpec((_G, d), lambda i, b: (i, 0)),
          pl.BlockSpec((_G, d), lambda i, b: (i % (t // _G), 0)),
          pl.BlockSpec((_G, 1), lambda i, b: (i, 0)),
          pl.BlockSpec(cb.shape, lambda i, b: (0, 0)),
          pl.BlockSpec(tids_f.shape, lambda i, b: (0, 0)),
          pl.BlockSpec(keys.shape, lambda i, b: (0, 0)),
          pl.BlockSpec(vals.shape, lambda i, b: (0, 0)),
      ],
      out_specs=pl.BlockSpec((_G, d), lambda i, b: (i, 0)),
      scratch_shapes=[
          pltpu.VMEM((_G // _P, w, d), jnp.float32),
          pltpu.VMEM((_G // _P, w, d), jnp.float32),
          pltpu.VMEM((_G, d), jnp.float32),
          pltpu.VMEM((_G, 128), jnp.float32),
      ],
  )
  return pl.pallas_call(
      _tc_body,
      grid_spec=spec,
      out_shape=jax.ShapeDtypeStruct((n, d), jnp.float32),
      compiler_params=pltpu.CompilerParams(
          dimension_semantics=("arbitrary",)),
  )(ids, emb, pe, ids2, cb, tids_f, keys, vals)


def kernel(input_ids, tok_emb, slot_keys, slot_values, centroid_codebook,
           pe, slot_tids):
  b, t = input_ids.shape
  d = tok_emb.shape[1]
  n = b * t
  ids = input_ids.reshape(n).astype(jnp.int32)
  tids_f = jnp.tile(
      slot_tids.reshape(_N_BUCKETS, _SPB), (1, 4)).astype(jnp.float32)

  emb = _sc_gather(ids, tok_emb)
  ids2 = ids.reshape(n, 1)
  out = _tc_combine(ids, emb, pe, ids2, centroid_codebook, tids_f,
                    slot_keys, slot_values, t)
  return out.reshape(b, t, d)


# speedup vs baseline: 9.2631x; 1.0334x over previous
import functools

import jax
import jax.numpy as jnp
from jax import lax
from jax.experimental import pallas as pl
from jax.experimental.pallas import tpu as pltpu
from jax.experimental.pallas import tpu_sc as plsc

_N_BUCKETS = 512
_SPB = 32
_TAU = 0.1
_ALPHA = 0.5

_G = 512
_P = 128


def _sc_gather(ids, tok_emb, tids2d, codebook):
  n = ids.shape[0]
  d = tok_emb.shape[1]
  info = plsc.get_sparse_core_info()
  nc, ns = info.num_cores, info.num_subcores
  nw = nc * ns
  per = n // nw
  ch = 128
  nch = per // ch

  mesh = plsc.VectorSubcoreMesh(core_axis_name="c", subcore_axis_name="s")

  @functools.partial(
      pl.kernel,
      out_type=(
          jax.ShapeDtypeStruct((n, d), jnp.float32),
          jax.ShapeDtypeStruct((n, d), jnp.float32),
          jax.ShapeDtypeStruct((n, 128), jnp.int32),
          jax.ShapeDtypeStruct((n,), jnp.int32),
      ),
      mesh=mesh,
      scratch_types=[
          pltpu.VMEM((nch, ch), jnp.int32),
          pltpu.VMEM((nch, ch), jnp.int32),
          pltpu.VMEM((per, d), jnp.float32),
          pltpu.VMEM((per, d), jnp.float32),
          pltpu.VMEM((per, 128), jnp.int32),
          pltpu.SemaphoreType.DMA,
      ],
  )
  def k(ids_hbm, emb_hbm, tids_hbm, cb_hbm,
        emb_o, anch_o, gt_o, bkt_o,
        ids_v, bkt_v, emb_v, anch_v, gt_v, sem):
    wid = lax.axis_index("s") * nc + lax.axis_index("c")
    base = wid * per
    for j in range(nch):
      pltpu.sync_copy(ids_hbm.at[pl.ds(base + j * ch, ch)], ids_v.at[j])
    for j in range(nch):
      for c in range(ch // 16):
        v = ids_v[j, pl.ds(c * 16, 16)]
        bkt_v[j, pl.ds(c * 16, 16)] = lax.rem(v, _N_BUCKETS)
    copies = []
    for j in range(nch):
      copies.append(pltpu.async_copy(
          emb_hbm.at[ids_v.at[j]], emb_v.at[pl.ds(j * ch, ch)], sem))
      copies.append(pltpu.async_copy(
          cb_hbm.at[bkt_v.at[j]], anch_v.at[pl.ds(j * ch, ch)], sem))
      copies.append(pltpu.async_copy(
          tids_hbm.at[bkt_v.at[j]], gt_v.at[pl.ds(j * ch, ch)], sem))
    for cp in copies:
      cp.wait()
    pltpu.sync_copy(emb_v, emb_o.at[pl.ds(base, per)])
    pltpu.sync_copy(anch_v, anch_o.at[pl.ds(base, per)])
    pltpu.sync_copy(gt_v, gt_o.at[pl.ds(base, per)])
    for j in range(nch):
      pltpu.sync_copy(bkt_v.at[j], bkt_o.at[pl.ds(base + j * ch, ch)])

  return k(ids, tok_emb, tids2d, codebook)


def _tc_body(bkt_ref, emb_ref, pe_ref, ids_ref, gt_ref, anc_ref,
             keys_ref, vals_ref, out_ref, kall, vall, u_ref):
  i = pl.program_id(0)
  h = emb_ref[...] + pe_ref[...]
  qn = h * lax.rsqrt(jnp.maximum(jnp.sum(h * h, -1, keepdims=True), 1e-24))
  u = _ALPHA * qn + (1.0 - _ALPHA) * anc_ref[...]
  u = u * lax.rsqrt(jnp.maximum(jnp.sum(u * u, -1, keepdims=True), 1e-24))
  u_ref[...] = u.astype(jnp.bfloat16)

  w = _P * _SPB
  ng = _G // _P
  col = lax.broadcasted_iota(jnp.int32, (_P, w), 1)
  row = lax.broadcasted_iota(jnp.int32, (_P, w), 0)
  bd = (col // _SPB) == row

  bs = [bkt_ref[i * _G + t] * _SPB for t in range(_G)]
  for g in range(ng):
    for j in range(_P):
      kall[g, pl.ds(j * _SPB, _SPB), :] = \
          keys_ref[pl.ds(bs[g * _P + j], _SPB), :]
  for g in range(ng):
    for j in range(_P):
      vall[g, pl.ds(j * _SPB, _SPB), :] = \
          vals_ref[pl.ds(bs[g * _P + j], _SPB), :]

  for g in range(ng):
    q = u_ref[pl.ds(g * _P, _P), :]
    scores = lax.dot_general(
        q, kall[g], (((1,), (1,)), ((), ())),
        preferred_element_type=jnp.float32)
    tcp = gt_ref[pl.ds(g * _P, _P), :]
    ttile = jnp.concatenate([tcp] * (w // 128), axis=1)
    idsp = ids_ref[pl.ds(g * _P, _P), :]
    match = jnp.where(bd & (ttile == idsp), 1.0, 0.0).astype(jnp.float32)
    msum = jnp.sum(match, -1, keepdims=True)
    sc = jnp.where(bd, scores * (1.0 / _TAU), -1e30)
    m = jnp.max(sc, -1, keepdims=True)
    e = jnp.exp(sc - m)
    psoft = e / jnp.sum(e, -1, keepdims=True)
    probs = jnp.where(msum > 0, match / (msum + 1e-9), psoft)
    val = lax.dot_general(
        probs.astype(jnp.bfloat16), vall[g], (((1,), (0,)), ((), ())),
        preferred_element_type=jnp.float32)
    out_ref[pl.ds(g * _P, _P), :] = val


def _tc_combine(buckets, emb, pe, ids2, gtids, anchors, keys, vals, t):
  n, d = emb.shape
  w = _P * _SPB
  grid = (n // _G,)
  spec = pltpu.PrefetchScalarGridSpec(
      num_scalar_prefetch=1,
      grid=grid,
      in_specs=[
          pl.BlockSpec((_G, d), lambda i, b: (i, 0)),
          pl.BlockSpec((_G, d), lambda i, b: (i % (t // _G), 0)),
          pl.BlockSpec((_G, 1), lambda i, b: (i, 0)),
          pl.BlockSpec((_G, 128), lambda i, b: (i, 0)),
          pl.BlockSpec((_G, d), lambda i, b: (i, 0)),
          pl.BlockSpec(keys.shape, lambda i, b: (0, 0)),
          pl.BlockSpec(vals.shape, lambda i, b: (0, 0)),
      ],
      out_specs=pl.BlockSpec((_G, d), lambda i, b: (i, 0)),
      scratch_shapes=[
          pltpu.VMEM((_G // _P, w, d), jnp.bfloat16),
          pltpu.VMEM((_G // _P, w, d), jnp.bfloat16),
          pltpu.VMEM((_G, d), jnp.bfloat16),
      ],
  )
  return pl.pallas_call(
      _tc_body,
      grid_spec=spec,
      out_shape=jax.ShapeDtypeStruct((n, d), jnp.float32),
      compiler_params=pltpu.CompilerParams(
          dimension_semantics=("arbitrary",)),
  )(buckets, emb, pe, ids2, gtids, anchors, keys, vals)


def kernel(input_ids, tok_emb, slot_keys, slot_values, centroid_codebook,
           pe, slot_tids):
  b, t = input_ids.shape
  d = tok_emb.shape[1]
  n = b * t
  ids = input_ids.reshape(n).astype(jnp.int32)
  tids_tiled = jnp.tile(
      slot_tids.astype(jnp.int32).reshape(_N_BUCKETS, _SPB), (1, 4))
  keys_bf = slot_keys.astype(jnp.bfloat16)
  vals_bf = slot_values.astype(jnp.bfloat16)

  emb, anchors, gtids, buckets = _sc_gather(ids, tok_emb, tids_tiled,
                                            centroid_codebook)
  ids2 = ids.reshape(n, 1)
  out = _tc_combine(buckets, emb, pe, ids2, gtids, anchors,
                    keys_bf, vals_bf, t)
  return out.reshape(b, t, d)


# speedup vs baseline: 9.4955x; 1.0251x over previous
import functools

import jax
import jax.numpy as jnp
from jax import lax
from jax.experimental import pallas as pl
from jax.experimental.pallas import tpu as pltpu
from jax.experimental.pallas import tpu_sc as plsc

_N_BUCKETS = 512
_SPB = 32
_TAU = 0.1
_ALPHA = 0.5

_G = 256
_P = 128


def _sc_gather(ids, tok_emb, tids2d, codebook):
  n = ids.shape[0]
  d = tok_emb.shape[1]
  info = plsc.get_sparse_core_info()
  nc, ns = info.num_cores, info.num_subcores
  nw = nc * ns
  per = n // nw
  ch = 128
  nch = per // ch

  mesh = plsc.VectorSubcoreMesh(core_axis_name="c", subcore_axis_name="s")

  @functools.partial(
      pl.kernel,
      out_type=(
          jax.ShapeDtypeStruct((n, d), jnp.float32),
          jax.ShapeDtypeStruct((n, d), jnp.float32),
          jax.ShapeDtypeStruct((n, 128), jnp.int32),
          jax.ShapeDtypeStruct((n,), jnp.int32),
      ),
      mesh=mesh,
      scratch_types=[
          pltpu.VMEM((nch, ch), jnp.int32),
          pltpu.VMEM((nch, ch), jnp.int32),
          pltpu.VMEM((per, d), jnp.float32),
          pltpu.VMEM((per, d), jnp.float32),
          pltpu.VMEM((per, 128), jnp.int32),
          pltpu.SemaphoreType.DMA,
      ],
  )
  def k(ids_hbm, emb_hbm, tids_hbm, cb_hbm,
        emb_o, anch_o, gt_o, bkt_o,
        ids_v, bkt_v, emb_v, anch_v, gt_v, sem):
    wid = lax.axis_index("s") * nc + lax.axis_index("c")
    base = wid * per
    for j in range(nch):
      pltpu.sync_copy(ids_hbm.at[pl.ds(base + j * ch, ch)], ids_v.at[j])
    for j in range(nch):
      for c in range(ch // 16):
        v = ids_v[j, pl.ds(c * 16, 16)]
        bkt_v[j, pl.ds(c * 16, 16)] = lax.rem(v, _N_BUCKETS)
    copies = []
    for j in range(nch):
      copies.append(pltpu.async_copy(
          emb_hbm.at[ids_v.at[j]], emb_v.at[pl.ds(j * ch, ch)], sem))
      copies.append(pltpu.async_copy(
          cb_hbm.at[bkt_v.at[j]], anch_v.at[pl.ds(j * ch, ch)], sem))
      copies.append(pltpu.async_copy(
          tids_hbm.at[bkt_v.at[j]], gt_v.at[pl.ds(j * ch, ch)], sem))
    for cp in copies:
      cp.wait()
    pltpu.sync_copy(emb_v, emb_o.at[pl.ds(base, per)])
    pltpu.sync_copy(anch_v, anch_o.at[pl.ds(base, per)])
    pltpu.sync_copy(gt_v, gt_o.at[pl.ds(base, per)])
    for j in range(nch):
      pltpu.sync_copy(bkt_v.at[j], bkt_o.at[pl.ds(base + j * ch, ch)])

  return k(ids, tok_emb, tids2d, codebook)


def _tc_body(bkt_ref, emb_ref, pe_ref, ids_ref, gt_ref, anc_ref,
             keys_ref, vals_ref, out_ref, kall, vall, u_ref):
  i = pl.program_id(0)
  h = emb_ref[...] + pe_ref[...]
  qn = h * lax.rsqrt(jnp.maximum(jnp.sum(h * h, -1, keepdims=True), 1e-24))
  u = _ALPHA * qn + (1.0 - _ALPHA) * anc_ref[...]
  u = u * lax.rsqrt(jnp.maximum(jnp.sum(u * u, -1, keepdims=True), 1e-24))
  u_ref[...] = u

  w = _P * _SPB
  ng = _G // _P
  col = lax.broadcasted_iota(jnp.int32, (_P, w), 1)
  row = lax.broadcasted_iota(jnp.int32, (_P, w), 0)
  bd = (col // _SPB) == row

  bs = [bkt_ref[i * _G + t] * _SPB for t in range(_G)]
  for g in range(ng):
    for j in range(_P):
      kall[g, pl.ds(j * _SPB, _SPB), :] = \
          keys_ref[pl.ds(bs[g * _P + j], _SPB), :]
  for g in range(ng):
    for j in range(_P):
      vall[g, pl.ds(j * _SPB, _SPB), :] = \
          vals_ref[pl.ds(bs[g * _P + j], _SPB), :]

  for g in range(ng):
    q = u_ref[pl.ds(g * _P, _P), :]
    scores = lax.dot_general(
        q, kall[g], (((1,), (1,)), ((), ())),
        preferred_element_type=jnp.float32)
    tcp = gt_ref[pl.ds(g * _P, _P), :]
    ttile = jnp.concatenate([tcp] * (w // 128), axis=1)
    idsp = ids_ref[pl.ds(g * _P, _P), :]
    match = jnp.where(bd & (ttile == idsp), 1.0, 0.0).astype(jnp.float32)
    msum = jnp.sum(match, -1, keepdims=True)
    sc = jnp.where(bd, scores * (1.0 / _TAU), -1e30)
    m = jnp.max(sc, -1, keepdims=True)
    e = jnp.exp(sc - m)
    psoft = e / jnp.sum(e, -1, keepdims=True)
    probs = jnp.where(msum > 0, match / (msum + 1e-9), psoft)
    val = lax.dot_general(
        probs, vall[g], (((1,), (0,)), ((), ())),
        preferred_element_type=jnp.float32)
    out_ref[pl.ds(g * _P, _P), :] = val


def _tc_combine(buckets, emb, pe, ids2, gtids, anchors, keys, vals, t):
  n, d = emb.shape
  w = _P * _SPB
  grid = (n // _G,)
  spec = pltpu.PrefetchScalarGridSpec(
      num_scalar_prefetch=1,
      grid=grid,
      in_specs=[
          pl.BlockSpec((_G, d), lambda i, b: (i, 0)),
          pl.BlockSpec((_G, d), lambda i, b: (i % (t // _G), 0)),
          pl.BlockSpec((_G, 1), lambda i, b: (i, 0)),
          pl.BlockSpec((_G, 128), lambda i, b: (i, 0)),
          pl.BlockSpec((_G, d), lambda i, b: (i, 0)),
          pl.BlockSpec(keys.shape, lambda i, b: (0, 0)),
          pl.BlockSpec(vals.shape, lambda i, b: (0, 0)),
      ],
      out_specs=pl.BlockSpec((_G, d), lambda i, b: (i, 0)),
      scratch_shapes=[
          pltpu.VMEM((_G // _P, w, d), jnp.float32),
          pltpu.VMEM((_G // _P, w, d), jnp.float32),
          pltpu.VMEM((_G, d), jnp.float32),
      ],
  )
  return pl.pallas_call(
      _tc_body,
      grid_spec=spec,
      out_shape=jax.ShapeDtypeStruct((n, d), jnp.float32),
      compiler_params=pltpu.CompilerParams(
          dimension_semantics=("arbitrary",)),
  )(buckets, emb, pe, ids2, gtids, anchors, keys, vals)


def kernel(input_ids, tok_emb, slot_keys, slot_values, centroid_codebook,
           pe, slot_tids):
  b, t = input_ids.shape
  d = tok_emb.shape[1]
  n = b * t
  ids = input_ids.reshape(n).astype(jnp.int32)
  tids_tiled = jnp.tile(
      slot_tids.astype(jnp.int32).reshape(_N_BUCKETS, _SPB), (1, 4))
  emb, anchors, gtids, buckets = _sc_gather(ids, tok_emb, tids_tiled,
                                            centroid_codebook)
  ids2 = ids.reshape(n, 1)
  out = _tc_combine(buckets, emb, pe, ids2, gtids, anchors,
                    slot_keys, slot_values, t)
  return out.reshape(b, t, d)


# speedup vs baseline: 10.1119x; 1.0649x over previous
import functools

import jax
import jax.numpy as jnp
from jax import lax
from jax.experimental import pallas as pl
from jax.experimental.pallas import tpu as pltpu
from jax.experimental.pallas import tpu_sc as plsc

_N_BUCKETS = 512
_SPB = 32
_TAU = 0.1
_ALPHA = 0.5

_G = 512
_P = 128


def _sc_gather(ids, tok_emb, tids2d, codebook):
  n = ids.shape[0]
  d = tok_emb.shape[1]
  info = plsc.get_sparse_core_info()
  nc, ns = info.num_cores, info.num_subcores
  nw = nc * ns
  per = n // nw
  ch = 128
  nch = per // ch

  mesh = plsc.VectorSubcoreMesh(core_axis_name="c", subcore_axis_name="s")

  @functools.partial(
      pl.kernel,
      out_type=(
          jax.ShapeDtypeStruct((n, d), jnp.float32),
          jax.ShapeDtypeStruct((n, d), jnp.float32),
          jax.ShapeDtypeStruct((n, 128), jnp.int32),
          jax.ShapeDtypeStruct((n,), jnp.int32),
      ),
      mesh=mesh,
      scratch_types=[
          pltpu.VMEM((nch, ch), jnp.int32),
          pltpu.VMEM((nch, ch), jnp.int32),
          pltpu.VMEM((per, d), jnp.float32),
          pltpu.VMEM((per, d), jnp.float32),
          pltpu.VMEM((per, 128), jnp.int32),
          pltpu.SemaphoreType.DMA,
          pltpu.SemaphoreType.DMA,
          pltpu.SemaphoreType.DMA,
          pltpu.SemaphoreType.DMA,
      ],
  )
  def k(ids_hbm, emb_hbm, tids_hbm, cb_hbm,
        emb_o, anch_o, gt_o, bkt_o,
        ids_v, bkt_v, emb_v, anch_v, gt_v, sem_e, sem_a, sem_g, sem_w):
    wid = lax.axis_index("s") * nc + lax.axis_index("c")
    base = wid * per
    for j in range(nch):
      pltpu.sync_copy(ids_hbm.at[pl.ds(base + j * ch, ch)], ids_v.at[j])
    for j in range(nch):
      for c in range(ch // 16):
        v = ids_v[j, pl.ds(c * 16, 16)]
        bkt_v[j, pl.ds(c * 16, 16)] = lax.rem(v, _N_BUCKETS)
    ce, ca, cg = [], [], []
    for j in range(nch):
      ce.append(pltpu.async_copy(
          emb_hbm.at[ids_v.at[j]], emb_v.at[pl.ds(j * ch, ch)], sem_e))
      ca.append(pltpu.async_copy(
          cb_hbm.at[bkt_v.at[j]], anch_v.at[pl.ds(j * ch, ch)], sem_a))
      cg.append(pltpu.async_copy(
          tids_hbm.at[bkt_v.at[j]], gt_v.at[pl.ds(j * ch, ch)], sem_g))
    wr = []
    for j in range(nch):
      wr.append(pltpu.async_copy(
          bkt_v.at[j], bkt_o.at[pl.ds(base + j * ch, ch)], sem_w))
    for cp in ce:
      cp.wait()
    wr.append(pltpu.async_copy(emb_v, emb_o.at[pl.ds(base, per)], sem_w))
    for cp in ca:
      cp.wait()
    wr.append(pltpu.async_copy(anch_v, anch_o.at[pl.ds(base, per)], sem_w))
    for cp in cg:
      cp.wait()
    wr.append(pltpu.async_copy(gt_v, gt_o.at[pl.ds(base, per)], sem_w))
    for cp in wr:
      cp.wait()

  return k(ids, tok_emb, tids2d, codebook)


def _tc_body(bkt_ref, emb_ref, pe_ref, ids_ref, gt_ref, anc_ref,
             keys_ref, vals_ref, out_ref, kall, vall, u_ref):
  i = pl.program_id(0)
  nst = pe_ref.shape[0] // _G
  h = emb_ref[...] + pe_ref[pl.ds((i % nst) * _G, _G), :]
  qn = h * lax.rsqrt(jnp.maximum(jnp.sum(h * h, -1, keepdims=True), 1e-24))
  u = _ALPHA * qn + (1.0 - _ALPHA) * anc_ref[...]
  u = u * lax.rsqrt(jnp.maximum(jnp.sum(u * u, -1, keepdims=True), 1e-24))
  u_ref[...] = u

  w = _P * _SPB
  ng = _G // _P
  col = lax.broadcasted_iota(jnp.int32, (_P, w), 1)
  row = lax.broadcasted_iota(jnp.int32, (_P, w), 0)
  bd = (col // _SPB) == row

  bs = [bkt_ref[i * _G + t] * _SPB for t in range(_G)]
  for g in range(ng):
    for j in range(_P):
      kall[g, pl.ds(j * _SPB, _SPB), :] = \
          keys_ref[pl.ds(bs[g * _P + j], _SPB), :]
  for g in range(ng):
    for j in range(_P):
      vall[g, pl.ds(j * _SPB, _SPB), :] = \
          vals_ref[pl.ds(bs[g * _P + j], _SPB), :]

  for g in range(ng):
    q = u_ref[pl.ds(g * _P, _P), :]
    scores = lax.dot_general(
        q, kall[g], (((1,), (1,)), ((), ())),
        preferred_element_type=jnp.float32)
    tcp = gt_ref[pl.ds(g * _P, _P), :]
    ttile = jnp.concatenate([tcp] * (w // 128), axis=1)
    idsp = ids_ref[pl.ds(g * _P, _P), :]
    match = jnp.where(bd & (ttile == idsp), 1.0, 0.0).astype(jnp.float32)
    msum = jnp.sum(match, -1, keepdims=True)
    sc = jnp.where(bd, scores * (1.0 / _TAU), -1e30)
    m = jnp.max(sc, -1, keepdims=True)
    e = jnp.exp(sc - m)
    psoft = e / jnp.sum(e, -1, keepdims=True)
    probs = jnp.where(msum > 0, match / (msum + 1e-9), psoft)
    val = lax.dot_general(
        probs, vall[g], (((1,), (0,)), ((), ())),
        preferred_element_type=jnp.float32)
    out_ref[pl.ds(g * _P, _P), :] = val


def _tc_combine(buckets, emb, pe, ids2, gtids, anchors, keys, vals, t):
  n, d = emb.shape
  w = _P * _SPB
  grid = (n // _G,)
  spec = pltpu.PrefetchScalarGridSpec(
      num_scalar_prefetch=1,
      grid=grid,
      in_specs=[
          pl.BlockSpec((_G, d), lambda i, b: (i, 0)),
          pl.BlockSpec((t, d), lambda i, b: (0, 0)),
          pl.BlockSpec((_G, 1), lambda i, b: (i, 0)),
          pl.BlockSpec((_G, 128), lambda i, b: (i, 0)),
          pl.BlockSpec((_G, d), lambda i, b: (i, 0)),
          pl.BlockSpec(keys.shape, lambda i, b: (0, 0)),
          pl.BlockSpec(vals.shape, lambda i, b: (0, 0)),
      ],
      out_specs=pl.BlockSpec((_G, d), lambda i, b: (i, 0)),
      scratch_shapes=[
          pltpu.VMEM((_G // _P, w, d), jnp.float32),
          pltpu.VMEM((_G // _P, w, d), jnp.float32),
          pltpu.VMEM((_G, d), jnp.float32),
      ],
  )
  return pl.pallas_call(
      _tc_body,
      grid_spec=spec,
      out_shape=jax.ShapeDtypeStruct((n, d), jnp.float32),
      compiler_params=pltpu.CompilerParams(
          dimension_semantics=("arbitrary",)),
  )(buckets, emb, pe, ids2, gtids, anchors, keys, vals)


def kernel(input_ids, tok_emb, slot_keys, slot_values, centroid_codebook,
           pe, slot_tids):
  b, t = input_ids.shape
  d = tok_emb.shape[1]
  n = b * t
  ids = input_ids.reshape(n).astype(jnp.int32)
  tids_tiled = jnp.tile(
      slot_tids.astype(jnp.int32).reshape(_N_BUCKETS, _SPB), (1, 4))
  emb, anchors, gtids, buckets = _sc_gather(ids, tok_emb, tids_tiled,
                                            centroid_codebook)
  ids2 = ids.reshape(n, 1)
  out = _tc_combine(buckets, emb, pe, ids2, gtids, anchors,
                    slot_keys, slot_values, t)
  return out.reshape(b, t, d)
